# initial kernel scaffold (unmeasured)
import jax
import jax.numpy as jnp
from jax import lax
from jax.experimental import pallas as pl
from jax.experimental.pallas import tpu as pltpu

D_HALF = 512
F = 4096
F_HALF = 2048


def kernel(x, dy):
    m, d = x.shape
    _, f = dy.shape

    def body(x_ref, dy_ref, out_ref, b_send, y_recv, x_recv,
             y_send_sem, y_recv_sem, x_send_sem, x_recv_sem):
        a = lax.axis_index("x")
        b = lax.axis_index("y")

        barrier = pltpu.get_barrier_semaphore()
        pl.semaphore_signal(barrier, inc=1, device_id=(1 - a, b),
                            device_id_type=pl.DeviceIdType.MESH)
        pl.semaphore_signal(barrier, inc=1, device_id=(a, 1 - b),
                            device_id_type=pl.DeviceIdType.MESH)
        pl.semaphore_wait(barrier, 2)

        def dot(p, q):
            return lax.dot_general(p, q, (((0,), (0,)), ((), ())),
                                   preferred_element_type=jnp.float32)

        x_own = x_ref[:, pl.ds(b * D_HALF, D_HALF)].astype(jnp.bfloat16)
        x_oth = x_ref[:, pl.ds((1 - b) * D_HALF, D_HALF)].astype(jnp.bfloat16)
        dy_a = dy_ref[:, pl.ds(a * F_HALF, F_HALF)].astype(jnp.bfloat16)
        dy_na = dy_ref[:, pl.ds((1 - a) * F_HALF, F_HALF)].astype(jnp.bfloat16)

        b_send[:, :] = dot(x_oth, dy_a).astype(jnp.bfloat16)

        rdma_y = pltpu.make_async_remote_copy(
            src_ref=b_send, dst_ref=y_recv,
            send_sem=y_send_sem, recv_sem=y_recv_sem,
            device_id=(a, 1 - b), device_id_type=pl.DeviceIdType.MESH,
        )
        rdma_y.start()

        out_ref[:, pl.ds(a * F_HALF, F_HALF)] = dot(x_own, dy_a)
        out_ref[:, pl.ds((1 - a) * F_HALF, F_HALF)] = dot(x_own, dy_na)

        rdma_y.wait()

        rdma_x = pltpu.make_async_remote_copy(
            src_ref=y_recv, dst_ref=x_recv,
            send_sem=x_send_sem, recv_sem=x_recv_sem,
            device_id=(1 - a, b), device_id_type=pl.DeviceIdType.MESH,
        )
        rdma_x.start()

        cols_a = pl.ds(a * F_HALF, F_HALF)
        out_ref[:, cols_a] = out_ref[:, cols_a] + y_recv[:, :].astype(jnp.float32)

        rdma_x.wait()

        cols_na = pl.ds((1 - a) * F_HALF, F_HALF)
        out_ref[:, cols_na] = out_ref[:, cols_na] + x_recv[:, :].astype(jnp.float32)

    return pl.pallas_call(
        body,
        out_shape=jax.ShapeDtypeStruct((D_HALF, f), jnp.float32),
        in_specs=[
            pl.BlockSpec(memory_space=pltpu.VMEM),
            pl.BlockSpec(memory_space=pltpu.VMEM),
        ],
        out_specs=pl.BlockSpec(memory_space=pltpu.VMEM),
        scratch_shapes=[
            pltpu.VMEM((D_HALF, F_HALF), jnp.bfloat16),
            pltpu.VMEM((D_HALF, F_HALF), jnp.bfloat16),
            pltpu.VMEM((D_HALF, F_HALF), jnp.bfloat16),
            pltpu.SemaphoreType.DMA,
            pltpu.SemaphoreType.DMA,
            pltpu.SemaphoreType.DMA,
            pltpu.SemaphoreType.DMA,
        ],
        compiler_params=pltpu.CompilerParams(collective_id=0),
    )(x, dy)


# baseline (device time: 71401 ns/iter reference)
import jax
import jax.numpy as jnp
from jax import lax
from jax.experimental import pallas as pl
from jax.experimental.pallas import tpu as pltpu

D_HALF = 512
F = 4096
F_HALF = 2048


def kernel(x, dy):
    m, d = x.shape
    _, f = dy.shape

    def body(x_ref, dy_ref, out_ref, b_send, y_recv, x_recv,
             y_send_sem, y_recv_sem, x_send_sem, x_recv_sem):
        a = lax.axis_index("x")
        b = lax.axis_index("y")

        barrier = pltpu.get_barrier_semaphore()
        pl.semaphore_signal(barrier, inc=1, device_id=(1 - a, b),
                            device_id_type=pl.DeviceIdType.MESH)
        pl.semaphore_signal(barrier, inc=1, device_id=(a, 1 - b),
                            device_id_type=pl.DeviceIdType.MESH)
        pl.semaphore_wait(barrier, 2)

        def dot(p, q):
            return lax.dot_general(p, q, (((0,), (0,)), ((), ())),
                                   preferred_element_type=jnp.float32)

        x_own = x_ref[:, pl.ds(b * D_HALF, D_HALF)].astype(jnp.bfloat16)
        x_oth = x_ref[:, pl.ds((1 - b) * D_HALF, D_HALF)].astype(jnp.bfloat16)
        dy_a = dy_ref[:, pl.ds(a * F_HALF, F_HALF)].astype(jnp.bfloat16)
        dy_na = dy_ref[:, pl.ds((1 - a) * F_HALF, F_HALF)].astype(jnp.bfloat16)

        b_send[:, :] = dot(x_oth, dy_a).astype(jnp.bfloat16)

        rdma_y = pltpu.make_async_remote_copy(
            src_ref=b_send, dst_ref=y_recv,
            send_sem=y_send_sem, recv_sem=y_recv_sem,
            device_id=(a, 1 - b), device_id_type=pl.DeviceIdType.MESH,
        )
        rdma_y.start()

        out_ref[:, pl.ds(a * F_HALF, F_HALF)] = dot(x_own, dy_a)
        out_ref[:, pl.ds((1 - a) * F_HALF, F_HALF)] = dot(x_own, dy_na)

        rdma_y.wait()

        rdma_x = pltpu.make_async_remote_copy(
            src_ref=y_recv, dst_ref=x_recv,
            send_sem=x_send_sem, recv_sem=x_recv_sem,
            device_id=(1 - a, b), device_id_type=pl.DeviceIdType.MESH,
        )
        rdma_x.start()

        cols_a = pl.ds(a * F_HALF, F_HALF)
        out_ref[:, cols_a] = out_ref[:, cols_a] + y_recv[:, :].astype(jnp.float32)

        rdma_x.wait()

        cols_na = pl.ds((1 - a) * F_HALF, F_HALF)
        out_ref[:, cols_na] = out_ref[:, cols_na] + x_recv[:, :].astype(jnp.float32)

    return pl.pallas_call(
        body,
        out_shape=jax.ShapeDtypeStruct((D_HALF, f), jnp.float32),
        in_specs=[
            pl.BlockSpec(memory_space=pltpu.VMEM),
            pl.BlockSpec(memory_space=pltpu.VMEM),
        ],
        out_specs=pl.BlockSpec(memory_space=pltpu.VMEM),
        scratch_shapes=[
            pltpu.VMEM((D_HALF, F_HALF), jnp.bfloat16),
            pltpu.VMEM((D_HALF, F_HALF), jnp.bfloat16),
            pltpu.VMEM((D_HALF, F_HALF), jnp.bfloat16),
            pltpu.SemaphoreType.DMA,
            pltpu.SemaphoreType.DMA,
            pltpu.SemaphoreType.DMA,
            pltpu.SemaphoreType.DMA,
        ],
        compiler_params=pltpu.CompilerParams(
            collective_id=0,
            vmem_limit_bytes=100 * 1024 * 1024,
        ),
    )(x, dy)


# device time: 53135 ns/iter; 1.3438x vs baseline; 1.3438x over previous
import functools

import jax
import jax.numpy as jnp
from jax import lax
from jax.experimental import pallas as pl
from jax.experimental.pallas import tpu as pltpu

D_HALF = 512
F_HALF = 2048
K = 16
C = F_HALF // K
KF = 4
CF = F_HALF // KF
R = K // KF


def kernel(x, dy):
    m, d = x.shape
    _, f = dy.shape

    def body(x_hbm, dy_hbm, out_ref,
             x_vmem, dy_vmem, b_send, y_recv, f_send, x_recv,
             x_sems, dy_sem,
             y_send_sem, y_recv_sem, x_send_sem, x_recv_sem):
        a = lax.axis_index("x")
        b = lax.axis_index("y")

        barrier = pltpu.get_barrier_semaphore()
        pl.semaphore_signal(barrier, inc=1, device_id=(1 - a, b),
                            device_id_type=pl.DeviceIdType.MESH)
        pl.semaphore_signal(barrier, inc=1, device_id=(a, 1 - b),
                            device_id_type=pl.DeviceIdType.MESH)
        pl.semaphore_wait(barrier, 2)

        x_dmas = []
        for h in range(2):
            which = (1 - b) if h == 0 else b
            dma = pltpu.make_async_copy(
                x_hbm.at[:, pl.ds(which * D_HALF, D_HALF)],
                x_vmem.at[:, pl.ds(h * D_HALF, D_HALF)],
                x_sems.at[h],
            )
            dma.start()
            x_dmas.append(dma)
        dy_dmas = []
        for j in range(KF):
            dma = pltpu.make_async_copy(
                dy_hbm.at[:, pl.ds(a * F_HALF + j * CF, CF)],
                dy_vmem.at[:, pl.ds(j * CF, CF)],
                dy_sem.at[j],
            )
            dma.start()
            dy_dmas.append(dma)

        def dot(p, q):
            return lax.dot_general(p, q, (((0,), (0,)), ((), ())),
                                   preferred_element_type=jnp.float32)

        def dy_chunk(k):
            return dy_vmem[:, pl.ds(k * C, C)].astype(jnp.bfloat16)

        x_dmas[0].wait()
        x_oth = x_vmem[:, :D_HALF].astype(jnp.bfloat16)

        rdma_y = []
        for k in range(K):
            if k % R == 0:
                dy_dmas[k // R].wait()
            b_send[k] = dot(x_oth, dy_chunk(k)).astype(jnp.bfloat16)
            r = pltpu.make_async_remote_copy(
                src_ref=b_send.at[k], dst_ref=y_recv.at[k],
                send_sem=y_send_sem.at[k], recv_sem=y_recv_sem.at[k],
                device_id=(a, 1 - b), device_id_type=pl.DeviceIdType.MESH,
            )
            r.start()
            rdma_y.append(r)

        x_dmas[1].wait()
        x_own = x_vmem[:, D_HALF:].astype(jnp.bfloat16)

        rdma_x = []
        for k in range(K):
            rdma_y[k].wait_recv()
            fin = dot(x_own, dy_chunk(k)) + y_recv[k].astype(jnp.float32)
            f_send[k] = fin.astype(jnp.bfloat16)
            r = pltpu.make_async_remote_copy(
                src_ref=f_send.at[k], dst_ref=x_recv.at[k],
                send_sem=x_send_sem.at[k], recv_sem=x_recv_sem.at[k],
                device_id=(1 - a, b), device_id_type=pl.DeviceIdType.MESH,
            )
            r.start()
            rdma_x.append(r)
            out_ref[:, pl.ds(a * F_HALF + k * C, C)] = fin

        for k in range(K):
            rdma_x[k].wait_recv()
            out_ref[:, pl.ds((1 - a) * F_HALF + k * C, C)] = (
                x_recv[k].astype(jnp.float32)
            )

        for k in range(K):
            rdma_y[k].wait_send()
            rdma_x[k].wait_send()

        @functools.partial(pl.run_scoped,
                           second_barrier=pltpu.SemaphoreType.REGULAR)
        def _(second_barrier):
            for nbr in [(1 - a, b), (a, 1 - b)]:
                pl.semaphore_signal(second_barrier, inc=1, device_id=nbr,
                                    device_id_type=pl.DeviceIdType.MESH)
            pl.semaphore_wait(second_barrier, 2)

    return pl.pallas_call(
        body,
        out_shape=jax.ShapeDtypeStruct((D_HALF, f), jnp.float32),
        in_specs=[
            pl.BlockSpec(memory_space=pl.ANY),
            pl.BlockSpec(memory_space=pl.ANY),
        ],
        out_specs=pl.BlockSpec(memory_space=pltpu.VMEM),
        scratch_shapes=[
            pltpu.VMEM((m, d), jnp.float32),
            pltpu.VMEM((m, F_HALF), jnp.float32),
            pltpu.VMEM((K, D_HALF, C), jnp.bfloat16),
            pltpu.VMEM((K, D_HALF, C), jnp.bfloat16),
            pltpu.VMEM((K, D_HALF, C), jnp.bfloat16),
            pltpu.VMEM((K, D_HALF, C), jnp.bfloat16),
            pltpu.SemaphoreType.DMA((2,)),
            pltpu.SemaphoreType.DMA((KF,)),
            pltpu.SemaphoreType.DMA((K,)),
            pltpu.SemaphoreType.DMA((K,)),
            pltpu.SemaphoreType.DMA((K,)),
            pltpu.SemaphoreType.DMA((K,)),
        ],
        compiler_params=pltpu.CompilerParams(
            collective_id=0,
            vmem_limit_bytes=100 * 1024 * 1024,
        ),
    )(x, dy)


# device time: 47234 ns/iter; 1.5116x vs baseline; 1.1249x over previous
import functools

import jax
import jax.numpy as jnp
from jax import lax
from jax.experimental import pallas as pl
from jax.experimental.pallas import tpu as pltpu

D_HALF = 512
F_HALF = 2048
K = 8
C = F_HALF // K
KF = 4
CF = F_HALF // KF
R = K // KF


def kernel(x, dy):
    m, d = x.shape
    _, f = dy.shape

    def body(x_hbm, dy_hbm, out_ref,
             x_vmem, dy_vmem, b_send, y_recv, f_send, x_recv,
             x_sems, dy_sem,
             y_send_sem, y_recv_sem, x_send_sem, x_recv_sem):
        a = lax.axis_index("x")
        b = lax.axis_index("y")

        barrier = pltpu.get_barrier_semaphore()
        pl.semaphore_signal(barrier, inc=1, device_id=(1 - a, b),
                            device_id_type=pl.DeviceIdType.MESH)
        pl.semaphore_signal(barrier, inc=1, device_id=(a, 1 - b),
                            device_id_type=pl.DeviceIdType.MESH)
        pl.semaphore_wait(barrier, 2)

        x_dmas = []
        for h in range(2):
            which = (1 - b) if h == 0 else b
            dma = pltpu.make_async_copy(
                x_hbm.at[:, pl.ds(which * D_HALF, D_HALF)],
                x_vmem.at[:, pl.ds(h * D_HALF, D_HALF)],
                x_sems.at[h],
            )
            dma.start()
            x_dmas.append(dma)
        dy_dmas = []
        for j in range(KF):
            dma = pltpu.make_async_copy(
                dy_hbm.at[:, pl.ds(a * F_HALF + j * CF, CF)],
                dy_vmem.at[:, pl.ds(j * CF, CF)],
                dy_sem.at[j],
            )
            dma.start()
            dy_dmas.append(dma)

        def dot(p, q):
            return lax.dot_general(p, q, (((0,), (0,)), ((), ())),
                                   preferred_element_type=jnp.float32)

        def dy_chunk(k):
            return dy_vmem[:, pl.ds(k * C, C)].astype(jnp.bfloat16)

        x_dmas[0].wait()
        x_oth = x_vmem[:, :D_HALF].astype(jnp.bfloat16)

        rdma_y = []
        for k in range(K):
            if k % R == 0:
                dy_dmas[k // R].wait()
            b_send[k] = dot(x_oth, dy_chunk(k)).astype(jnp.bfloat16)
            r = pltpu.make_async_remote_copy(
                src_ref=b_send.at[k], dst_ref=y_recv.at[k],
                send_sem=y_send_sem.at[k], recv_sem=y_recv_sem.at[k],
                device_id=(a, 1 - b), device_id_type=pl.DeviceIdType.MESH,
            )
            r.start()
            rdma_y.append(r)

        x_dmas[1].wait()
        x_own = x_vmem[:, D_HALF:].astype(jnp.bfloat16)

        rdma_x = []
        for k in range(K):
            rdma_y[k].wait_recv()
            fin = dot(x_own, dy_chunk(k)) + y_recv[k].astype(jnp.float32)
            f_send[k] = fin.astype(jnp.bfloat16)
            r = pltpu.make_async_remote_copy(
                src_ref=f_send.at[k], dst_ref=x_recv.at[k],
                send_sem=x_send_sem.at[k], recv_sem=x_recv_sem.at[k],
                device_id=(1 - a, b), device_id_type=pl.DeviceIdType.MESH,
            )
            r.start()
            rdma_x.append(r)
            out_ref[:, pl.ds(a * F_HALF + k * C, C)] = fin

        for k in range(K):
            rdma_x[k].wait_recv()
            out_ref[:, pl.ds((1 - a) * F_HALF + k * C, C)] = (
                x_recv[k].astype(jnp.float32)
            )

        for k in range(K):
            rdma_y[k].wait_send()
            rdma_x[k].wait_send()

        @functools.partial(pl.run_scoped,
                           second_barrier=pltpu.SemaphoreType.REGULAR)
        def _(second_barrier):
            for nbr in [(1 - a, b), (a, 1 - b)]:
                pl.semaphore_signal(second_barrier, inc=1, device_id=nbr,
                                    device_id_type=pl.DeviceIdType.MESH)
            pl.semaphore_wait(second_barrier, 2)

    return pl.pallas_call(
        body,
        out_shape=jax.ShapeDtypeStruct((D_HALF, f), jnp.float32),
        in_specs=[
            pl.BlockSpec(memory_space=pl.ANY),
            pl.BlockSpec(memory_space=pl.ANY),
        ],
        out_specs=pl.BlockSpec(memory_space=pltpu.VMEM),
        scratch_shapes=[
            pltpu.VMEM((m, d), jnp.float32),
            pltpu.VMEM((m, F_HALF), jnp.float32),
            pltpu.VMEM((K, D_HALF, C), jnp.bfloat16),
            pltpu.VMEM((K, D_HALF, C), jnp.bfloat16),
            pltpu.VMEM((K, D_HALF, C), jnp.bfloat16),
            pltpu.VMEM((K, D_HALF, C), jnp.bfloat16),
            pltpu.SemaphoreType.DMA((2,)),
            pltpu.SemaphoreType.DMA((KF,)),
            pltpu.SemaphoreType.DMA((K,)),
            pltpu.SemaphoreType.DMA((K,)),
            pltpu.SemaphoreType.DMA((K,)),
            pltpu.SemaphoreType.DMA((K,)),
        ],
        compiler_params=pltpu.CompilerParams(
            collective_id=0,
            vmem_limit_bytes=100 * 1024 * 1024,
        ),
    )(x, dy)
